# baseline (speedup 1.0000x reference)
import functools

import jax
import jax.numpy as jnp
from jax import lax
from jax.experimental import pallas as pl
from jax.experimental.pallas import tpu as pltpu
from jax.experimental.pallas import tpu_sc as plsc

D_MODEL = 1024
D_HID = 512
N_TILES = 64
N_TOK = 8192

TB = 128
PADN = N_TOK + N_TILES * TB
NB = PADN // TB

NC = 2
NS = 16
NW = NC * NS
_SC_MESH = dict(core_axis_name="c", subcore_axis_name="s",
                num_cores=NC, num_subcores=NS)

GCH = 32


def _sc_gather(gidx, src, n_rows, width=D_MODEL):
    n_per_w = n_rows // NW
    n_chunks = n_per_w // GCH

    @functools.partial(
        pl.kernel,
        mesh=plsc.VectorSubcoreMesh(**_SC_MESH),
        out_type=jax.ShapeDtypeStruct((n_rows, width), jnp.float32),
        scratch_types=[
            pltpu.VMEM((GCH,), jnp.int32),
            pltpu.VMEM((GCH, width), jnp.float32),
            pltpu.SemaphoreType.DMA,
        ],
    )
    def gather_k(gidx_hbm, src_hbm, out_hbm, idx_v, rows_v, sem):
        wid = lax.axis_index("s") * NC + lax.axis_index("c")
        base = wid * n_per_w

        def body(c, carry):
            off = base + c * GCH
            pltpu.sync_copy(gidx_hbm.at[pl.ds(off, GCH)], idx_v)
            pltpu.async_copy(src_hbm.at[idx_v], rows_v, sem).wait()
            pltpu.sync_copy(rows_v, out_hbm.at[pl.ds(off, GCH)])
            return carry

        lax.fori_loop(0, n_chunks, body, 0)

    return gather_k(gidx, src)


def _ffn_block(be_ref, hp_ref, uw_ref, dw_ref, us_ref, dso_ref, out_ref,
               uws_ref, dws_ref):
    b = pl.program_id(0)
    valid = be_ref[1, b] == 1
    changed = jnp.logical_and(
        valid, jnp.logical_or(b == 0,
                              be_ref[0, b] != be_ref[0, jnp.maximum(b - 1, 0)]))

    @pl.when(changed)
    def _ternarize():
        uws_ref[...] = jnp.sign(uw_ref[0]).astype(jnp.bfloat16)
        dws_ref[...] = jnp.sign(dw_ref[0]).astype(jnp.bfloat16)

    @pl.when(valid)
    def _compute():
        hb = hp_ref[...].astype(jnp.bfloat16)
        hid = lax.dot_general(hb, uws_ref[...], (((1,), (1,)), ((), ())),
                              preferred_element_type=jnp.float32)
        hid = jnp.maximum(hid * us_ref[0], 0.0).astype(jnp.bfloat16)
        o = lax.dot_general(hid, dws_ref[...], (((1,), (1,)), ((), ())),
                            preferred_element_type=jnp.float32)
        out_ref[...] = o * dso_ref[0]


def _grouped_ffn(block_expert, hp, up_w, down_w, up_scale, dso):
    grid_spec = pltpu.PrefetchScalarGridSpec(
        num_scalar_prefetch=1,
        grid=(NB,),
        in_specs=[
            pl.BlockSpec((TB, D_MODEL), lambda b, be: (b, 0)),
            pl.BlockSpec((1, D_HID, D_MODEL), lambda b, be: (be[0, b], 0, 0)),
            pl.BlockSpec((1, D_MODEL, D_HID), lambda b, be: (be[0, b], 0, 0)),
            pl.BlockSpec((1, 1, D_HID), lambda b, be: (be[0, b], 0, 0)),
            pl.BlockSpec((1, 1, D_MODEL), lambda b, be: (be[0, b], 0, 0)),
        ],
        out_specs=pl.BlockSpec((TB, D_MODEL), lambda b, be: (b, 0)),
        scratch_shapes=[
            pltpu.VMEM((D_HID, D_MODEL), jnp.bfloat16),
            pltpu.VMEM((D_MODEL, D_HID), jnp.bfloat16),
        ],
    )
    return pl.pallas_call(
        _ffn_block,
        grid_spec=grid_spec,
        out_shape=jax.ShapeDtypeStruct((PADN, D_MODEL), jnp.float32),
    )(block_expert, hp, up_w, down_w, up_scale, dso)


def kernel(x, up_w, down_w, up_scale, down_scale, output_scale, norm_weight):
    h = x / jnp.sqrt(jnp.mean(x * x, axis=-1, keepdims=True) + 1e-6) * norm_weight
    sigs = jnp.sign(jnp.sign(up_w).sum(axis=1))
    scores = h @ sigs.T
    assign = jnp.argmax(scores, axis=-1).astype(jnp.int32)

    iota_n = jnp.arange(N_TOK, dtype=jnp.int32)
    oh = (assign[:, None] == jnp.arange(N_TILES, dtype=jnp.int32)[None, :])
    running = jnp.cumsum(oh.astype(jnp.int32), axis=0)
    counts = running[-1]
    ranks = jnp.sum(jnp.where(oh, running, 0), axis=1) - 1
    blocks_per_e = (counts + TB - 1) // TB
    pad_sizes = blocks_per_e * TB
    pstart = jnp.concatenate([jnp.zeros((1,), jnp.int32),
                              jnp.cumsum(pad_sizes)[:-1].astype(jnp.int32)])
    dst = pstart[assign] + ranks
    gidx0 = jnp.zeros((PADN,), jnp.int32).at[dst].set(iota_n)
    tok2p = dst
    cb = jnp.cumsum(blocks_per_e).astype(jnp.int32)
    raw_be = jnp.searchsorted(cb, jnp.arange(NB, dtype=jnp.int32),
                              side="right").astype(jnp.int32)
    block_expert = jnp.stack([jnp.minimum(raw_be, N_TILES - 1),
                              (raw_be < N_TILES).astype(jnp.int32)])
    iota_p = jnp.arange(PADN, dtype=jnp.int32)
    e_p = jnp.repeat(block_expert[0], TB)
    covered = (iota_p - pstart[e_p] < counts[e_p]) & (raw_be < N_TILES)[iota_p // TB]
    gidx = jnp.where(covered, gidx0, iota_p % N_TOK)

    hp = _sc_gather(gidx, h, PADN)
    us3 = up_scale[:, None, :]
    dso = (down_scale * output_scale[:, None])[:, None, :]
    op = _grouped_ffn(block_expert, hp, up_w, down_w, us3, dso)
    tile_out = _sc_gather(tok2p, op, N_TOK)
    return x + tile_out


# speedup vs baseline: 1.4072x; 1.4072x over previous
import functools

import jax
import jax.numpy as jnp
from jax import lax
from jax.experimental import pallas as pl
from jax.experimental.pallas import tpu as pltpu
from jax.experimental.pallas import tpu_sc as plsc

D_MODEL = 1024
D_HID = 512
N_TILES = 64
N_TOK = 8192

TB = 128
PADN = N_TOK + N_TILES * TB
NB = PADN // TB

NC = 2
NS = 16
NW = NC * NS
_SC_MESH = dict(core_axis_name="c", subcore_axis_name="s",
                num_cores=NC, num_subcores=NS)

GCH = 32


def _sc_gather(gidx, src, n_rows, width=D_MODEL):
    n_per_w = n_rows // NW
    n_chunks = n_per_w // GCH

    @functools.partial(
        pl.kernel,
        mesh=plsc.VectorSubcoreMesh(**_SC_MESH),
        out_type=jax.ShapeDtypeStruct((n_rows, width), jnp.float32),
        scratch_types=[
            pltpu.VMEM((GCH,), jnp.int32),
            pltpu.VMEM((GCH, width), jnp.float32),
            pltpu.SemaphoreType.DMA,
        ],
    )
    def gather_k(gidx_hbm, src_hbm, out_hbm, idx_v, rows_v, sem):
        wid = lax.axis_index("s") * NC + lax.axis_index("c")
        base = wid * n_per_w

        def body(c, carry):
            off = base + c * GCH
            pltpu.sync_copy(gidx_hbm.at[pl.ds(off, GCH)], idx_v)
            pltpu.async_copy(src_hbm.at[idx_v], rows_v, sem).wait()
            pltpu.sync_copy(rows_v, out_hbm.at[pl.ds(off, GCH)])
            return carry

        lax.fori_loop(0, n_chunks, body, 0)

    return gather_k(gidx, src)


def _ffn_block(be_ref, hp_ref, uw_ref, dw_ref, us_ref, dso_ref, out_ref,
               uws_ref, dws_ref):
    b = pl.program_id(0)
    valid = be_ref[1, b] == 1
    changed = jnp.logical_and(
        valid, jnp.logical_or(b == 0,
                              be_ref[0, b] != be_ref[0, jnp.maximum(b - 1, 0)]))

    @pl.when(changed)
    def _ternarize():
        uws_ref[...] = jnp.sign(uw_ref[0]).astype(jnp.bfloat16)
        dws_ref[...] = jnp.sign(dw_ref[0]).astype(jnp.bfloat16)

    @pl.when(valid)
    def _compute():
        hb = hp_ref[...].astype(jnp.bfloat16)
        hid = lax.dot_general(hb, uws_ref[...], (((1,), (1,)), ((), ())),
                              preferred_element_type=jnp.float32)
        hid = jnp.maximum(hid * us_ref[0], 0.0).astype(jnp.bfloat16)
        o = lax.dot_general(hid, dws_ref[...], (((1,), (1,)), ((), ())),
                            preferred_element_type=jnp.float32)
        out_ref[...] = o * dso_ref[0]


def _grouped_ffn(block_expert, hp, up_w, down_w, up_scale, dso):
    grid_spec = pltpu.PrefetchScalarGridSpec(
        num_scalar_prefetch=1,
        grid=(NB,),
        in_specs=[
            pl.BlockSpec((TB, D_MODEL), lambda b, be: (b, 0)),
            pl.BlockSpec((1, D_HID, D_MODEL), lambda b, be: (be[0, b], 0, 0)),
            pl.BlockSpec((1, D_MODEL, D_HID), lambda b, be: (be[0, b], 0, 0)),
            pl.BlockSpec((1, 1, D_HID), lambda b, be: (be[0, b], 0, 0)),
            pl.BlockSpec((1, 1, D_MODEL), lambda b, be: (be[0, b], 0, 0)),
        ],
        out_specs=pl.BlockSpec((TB, D_MODEL), lambda b, be: (b, 0)),
        scratch_shapes=[
            pltpu.VMEM((D_HID, D_MODEL), jnp.bfloat16),
            pltpu.VMEM((D_MODEL, D_HID), jnp.bfloat16),
        ],
    )
    return pl.pallas_call(
        _ffn_block,
        grid_spec=grid_spec,
        out_shape=jax.ShapeDtypeStruct((PADN, D_MODEL), jnp.float32),
    )(block_expert, hp, up_w, down_w, up_scale, dso)


def kernel(x, up_w, down_w, up_scale, down_scale, output_scale, norm_weight):
    h = x / jnp.sqrt(jnp.mean(x * x, axis=-1, keepdims=True) + 1e-6) * norm_weight
    sigs = jnp.sign(jnp.sign(up_w).sum(axis=1))
    scores = h @ sigs.T
    assign = jnp.argmax(scores, axis=-1).astype(jnp.int32)

    iota_n = jnp.arange(N_TOK, dtype=jnp.int32)
    oh = (assign[:, None] == jnp.arange(N_TILES, dtype=jnp.int32)[None, :])
    running = jnp.cumsum(oh.astype(jnp.int32), axis=0)
    counts = running[-1]
    ranks = jnp.take_along_axis(running, assign[:, None], axis=1)[:, 0] - 1
    blocks_per_e = (counts + TB - 1) // TB
    pad_sizes = blocks_per_e * TB
    pstart = jnp.concatenate([jnp.zeros((1,), jnp.int32),
                              jnp.cumsum(pad_sizes)[:-1].astype(jnp.int32)])
    dst = pstart[assign] + ranks
    gidx0 = jnp.zeros((PADN,), jnp.int32).at[dst].set(iota_n + 1)
    tok2p = dst
    cb = jnp.cumsum(blocks_per_e).astype(jnp.int32)
    raw_be = jnp.searchsorted(cb, jnp.arange(NB, dtype=jnp.int32),
                              side="right").astype(jnp.int32)
    block_expert = jnp.stack([jnp.minimum(raw_be, N_TILES - 1),
                              (raw_be < N_TILES).astype(jnp.int32)])
    iota_p = jnp.arange(PADN, dtype=jnp.int32)
    gidx = jnp.where(gidx0 == 0, iota_p % N_TOK, gidx0 - 1)

    hp = _sc_gather(gidx, h, PADN)
    us3 = up_scale[:, None, :]
    dso = (down_scale * output_scale[:, None])[:, None, :]
    op = _grouped_ffn(block_expert, hp, up_w, down_w, us3, dso)
    tile_out = _sc_gather(tok2p, op, N_TOK)
    return x + tile_out


# speedup vs baseline: 1.4409x; 1.0239x over previous
import functools

import jax
import jax.numpy as jnp
from jax import lax
from jax.experimental import pallas as pl
from jax.experimental.pallas import tpu as pltpu
from jax.experimental.pallas import tpu_sc as plsc

D_MODEL = 1024
D_HID = 512
N_TILES = 64
N_TOK = 8192

TB = 256
PADN = N_TOK + N_TILES * TB
NB = PADN // TB

NC = 2
NS = 16
NW = NC * NS
_SC_MESH = dict(core_axis_name="c", subcore_axis_name="s",
                num_cores=NC, num_subcores=NS)

GCH = 32


def _sc_gather(gidx, src, n_rows, width=D_MODEL):
    n_per_w = n_rows // NW
    n_chunks = n_per_w // GCH

    @functools.partial(
        pl.kernel,
        mesh=plsc.VectorSubcoreMesh(**_SC_MESH),
        out_type=jax.ShapeDtypeStruct((n_rows, width), jnp.float32),
        scratch_types=[
            pltpu.VMEM((GCH,), jnp.int32),
            pltpu.VMEM((GCH, width), jnp.float32),
            pltpu.SemaphoreType.DMA,
        ],
    )
    def gather_k(gidx_hbm, src_hbm, out_hbm, idx_v, rows_v, sem):
        wid = lax.axis_index("s") * NC + lax.axis_index("c")
        base = wid * n_per_w

        def body(c, carry):
            off = base + c * GCH
            pltpu.sync_copy(gidx_hbm.at[pl.ds(off, GCH)], idx_v)
            pltpu.async_copy(src_hbm.at[idx_v], rows_v, sem).wait()
            pltpu.sync_copy(rows_v, out_hbm.at[pl.ds(off, GCH)])
            return carry

        lax.fori_loop(0, n_chunks, body, 0)

    return gather_k(gidx, src)


def _ffn_block(be_ref, hp_ref, uw_ref, dw_ref, us_ref, dso_ref, out_ref,
               uws_ref, dws_ref):
    b = pl.program_id(0)
    valid = be_ref[1, b] == 1
    changed = jnp.logical_and(
        valid, jnp.logical_or(b == 0,
                              be_ref[0, b] != be_ref[0, jnp.maximum(b - 1, 0)]))

    @pl.when(changed)
    def _ternarize():
        uws_ref[...] = jnp.sign(uw_ref[0]).astype(jnp.bfloat16)
        dws_ref[...] = jnp.sign(dw_ref[0]).astype(jnp.bfloat16)

    @pl.when(valid)
    def _compute():
        hb = hp_ref[...].astype(jnp.bfloat16)
        hid = lax.dot_general(hb, uws_ref[...], (((1,), (1,)), ((), ())),
                              preferred_element_type=jnp.float32)
        hid = jnp.maximum(hid * us_ref[0], 0.0).astype(jnp.bfloat16)
        o = lax.dot_general(hid, dws_ref[...], (((1,), (1,)), ((), ())),
                            preferred_element_type=jnp.float32)
        out_ref[...] = o * dso_ref[0]


def _grouped_ffn(block_expert, hp, up_w, down_w, up_scale, dso):
    grid_spec = pltpu.PrefetchScalarGridSpec(
        num_scalar_prefetch=1,
        grid=(NB,),
        in_specs=[
            pl.BlockSpec((TB, D_MODEL), lambda b, be: (b, 0)),
            pl.BlockSpec((1, D_HID, D_MODEL), lambda b, be: (be[0, b], 0, 0)),
            pl.BlockSpec((1, D_MODEL, D_HID), lambda b, be: (be[0, b], 0, 0)),
            pl.BlockSpec((1, 1, D_HID), lambda b, be: (be[0, b], 0, 0)),
            pl.BlockSpec((1, 1, D_MODEL), lambda b, be: (be[0, b], 0, 0)),
        ],
        out_specs=pl.BlockSpec((TB, D_MODEL), lambda b, be: (b, 0)),
        scratch_shapes=[
            pltpu.VMEM((D_HID, D_MODEL), jnp.bfloat16),
            pltpu.VMEM((D_MODEL, D_HID), jnp.bfloat16),
        ],
    )
    return pl.pallas_call(
        _ffn_block,
        grid_spec=grid_spec,
        out_shape=jax.ShapeDtypeStruct((PADN, D_MODEL), jnp.float32),
    )(block_expert, hp, up_w, down_w, up_scale, dso)


def kernel(x, up_w, down_w, up_scale, down_scale, output_scale, norm_weight):
    h = x / jnp.sqrt(jnp.mean(x * x, axis=-1, keepdims=True) + 1e-6) * norm_weight
    sigs = jnp.sign(jnp.sign(up_w).sum(axis=1))
    scores = h @ sigs.T
    assign = jnp.argmax(scores, axis=-1).astype(jnp.int32)

    iota_n = jnp.arange(N_TOK, dtype=jnp.int32)
    oh = (assign[:, None] == jnp.arange(N_TILES, dtype=jnp.int32)[None, :])
    running = jnp.cumsum(oh.astype(jnp.int32), axis=0)
    counts = running[-1]
    ranks = jnp.take_along_axis(running, assign[:, None], axis=1)[:, 0] - 1
    blocks_per_e = (counts + TB - 1) // TB
    pad_sizes = blocks_per_e * TB
    pstart = jnp.concatenate([jnp.zeros((1,), jnp.int32),
                              jnp.cumsum(pad_sizes)[:-1].astype(jnp.int32)])
    dst = pstart[assign] + ranks
    gidx0 = jnp.zeros((PADN,), jnp.int32).at[dst].set(iota_n + 1)
    tok2p = dst
    cb = jnp.cumsum(blocks_per_e).astype(jnp.int32)
    raw_be = jnp.searchsorted(cb, jnp.arange(NB, dtype=jnp.int32),
                              side="right").astype(jnp.int32)
    block_expert = jnp.stack([jnp.minimum(raw_be, N_TILES - 1),
                              (raw_be < N_TILES).astype(jnp.int32)])
    iota_p = jnp.arange(PADN, dtype=jnp.int32)
    gidx = jnp.where(gidx0 == 0, iota_p % N_TOK, gidx0 - 1)

    hp = _sc_gather(gidx, h, PADN)
    us3 = up_scale[:, None, :]
    dso = (down_scale * output_scale[:, None])[:, None, :]
    op = _grouped_ffn(block_expert, hp, up_w, down_w, us3, dso)
    tile_out = _sc_gather(tok2p, op, N_TOK)
    return x + tile_out


# speedup vs baseline: 1.7339x; 1.2034x over previous
import functools

import jax
import jax.numpy as jnp
from jax import lax
from jax.experimental import pallas as pl
from jax.experimental.pallas import tpu as pltpu
from jax.experimental.pallas import tpu_sc as plsc

D_MODEL = 1024
D_HID = 512
N_TILES = 64
N_TOK = 8192

TB = 256
PADN = N_TOK + N_TILES * TB
NB = PADN // TB

NC = 2
NS = 16
NW = NC * NS
_SC_MESH = dict(core_axis_name="c", subcore_axis_name="s",
                num_cores=NC, num_subcores=NS)

GCH = 32


def _sc_gather(gidx, src, n_rows, width=D_MODEL):
    n_per_w = n_rows // NW
    n_chunks = n_per_w // GCH

    @functools.partial(
        pl.kernel,
        mesh=plsc.VectorSubcoreMesh(**_SC_MESH),
        out_type=jax.ShapeDtypeStruct((n_rows, width), jnp.float32),
        scratch_types=[
            pltpu.VMEM((GCH,), jnp.int32),
            pltpu.VMEM((GCH, width), jnp.float32),
            pltpu.SemaphoreType.DMA,
        ],
    )
    def gather_k(gidx_hbm, src_hbm, out_hbm, idx_v, rows_v, sem):
        wid = lax.axis_index("s") * NC + lax.axis_index("c")
        base = wid * n_per_w

        def body(c, carry):
            off = base + c * GCH
            pltpu.sync_copy(gidx_hbm.at[pl.ds(off, GCH)], idx_v)
            pltpu.async_copy(src_hbm.at[idx_v], rows_v, sem).wait()
            pltpu.sync_copy(rows_v, out_hbm.at[pl.ds(off, GCH)])
            return carry

        lax.fori_loop(0, n_chunks, body, 0)

    return gather_k(gidx, src)


def _sc_scatter_rows(dst, src, n_out):
    n_per_w = N_TOK // NW
    n_chunks = n_per_w // GCH

    @functools.partial(
        pl.kernel,
        mesh=plsc.VectorSubcoreMesh(**_SC_MESH),
        out_type=jax.ShapeDtypeStruct((n_out, D_MODEL), jnp.float32),
        scratch_types=[
            pltpu.VMEM((GCH,), jnp.int32),
            pltpu.VMEM((GCH, D_MODEL), jnp.float32),
            pltpu.SemaphoreType.DMA,
        ],
    )
    def scatter_k(dst_hbm, src_hbm, out_hbm, idx_v, rows_v, sem):
        wid = lax.axis_index("s") * NC + lax.axis_index("c")
        base = wid * n_per_w

        def body(c, carry):
            off = base + c * GCH
            pltpu.sync_copy(dst_hbm.at[pl.ds(off, GCH)], idx_v)
            pltpu.sync_copy(src_hbm.at[pl.ds(off, GCH)], rows_v)
            pltpu.async_copy(rows_v, out_hbm.at[idx_v], sem).wait()
            return carry

        lax.fori_loop(0, n_chunks, body, 0)

    return scatter_k(dst, src)


def _ffn_block(be_ref, hp_ref, uw_ref, dw_ref, us_ref, dso_ref, out_ref,
               uws_ref, dws_ref):
    b = pl.program_id(0)
    valid = be_ref[1, b] == 1
    changed = jnp.logical_and(
        valid, jnp.logical_or(b == 0,
                              be_ref[0, b] != be_ref[0, jnp.maximum(b - 1, 0)]))

    @pl.when(changed)
    def _ternarize():
        uws_ref[...] = jnp.sign(uw_ref[0]).astype(jnp.bfloat16)
        dws_ref[...] = jnp.sign(dw_ref[0]).astype(jnp.bfloat16)

    @pl.when(valid)
    def _compute():
        hb = hp_ref[...].astype(jnp.bfloat16)
        hid = lax.dot_general(hb, uws_ref[...], (((1,), (1,)), ((), ())),
                              preferred_element_type=jnp.float32)
        hid = jnp.maximum(hid * us_ref[0], 0.0).astype(jnp.bfloat16)
        o = lax.dot_general(hid, dws_ref[...], (((1,), (1,)), ((), ())),
                            preferred_element_type=jnp.float32)
        out_ref[...] = o * dso_ref[0]


def _grouped_ffn(block_expert, hp, up_w, down_w, up_scale, dso):
    grid_spec = pltpu.PrefetchScalarGridSpec(
        num_scalar_prefetch=1,
        grid=(NB,),
        in_specs=[
            pl.BlockSpec((TB, D_MODEL), lambda b, be: (b, 0)),
            pl.BlockSpec((1, D_HID, D_MODEL), lambda b, be: (be[0, b], 0, 0)),
            pl.BlockSpec((1, D_MODEL, D_HID), lambda b, be: (be[0, b], 0, 0)),
            pl.BlockSpec((1, 1, D_HID), lambda b, be: (be[0, b], 0, 0)),
            pl.BlockSpec((1, 1, D_MODEL), lambda b, be: (be[0, b], 0, 0)),
        ],
        out_specs=pl.BlockSpec((TB, D_MODEL), lambda b, be: (b, 0)),
        scratch_shapes=[
            pltpu.VMEM((D_HID, D_MODEL), jnp.bfloat16),
            pltpu.VMEM((D_MODEL, D_HID), jnp.bfloat16),
        ],
    )
    return pl.pallas_call(
        _ffn_block,
        grid_spec=grid_spec,
        out_shape=jax.ShapeDtypeStruct((PADN, D_MODEL), jnp.float32),
    )(block_expert, hp, up_w, down_w, up_scale, dso)


def kernel(x, up_w, down_w, up_scale, down_scale, output_scale, norm_weight):
    h = x / jnp.sqrt(jnp.mean(x * x, axis=-1, keepdims=True) + 1e-6) * norm_weight
    sigs = jnp.sign(jnp.sign(up_w).sum(axis=1))
    scores = h @ sigs.T
    assign = jnp.argmax(scores, axis=-1).astype(jnp.int32)

    iota_n = jnp.arange(N_TOK, dtype=jnp.int32)
    oh = (assign[:, None] == jnp.arange(N_TILES, dtype=jnp.int32)[None, :])
    running = jnp.cumsum(oh.astype(jnp.int32), axis=0)
    counts = running[-1]
    ranks = jnp.take_along_axis(running, assign[:, None], axis=1)[:, 0] - 1
    blocks_per_e = (counts + TB - 1) // TB
    pad_sizes = blocks_per_e * TB
    pstart = jnp.concatenate([jnp.zeros((1,), jnp.int32),
                              jnp.cumsum(pad_sizes)[:-1].astype(jnp.int32)])
    dst = pstart[assign] + ranks
    tok2p = dst
    cb = jnp.cumsum(blocks_per_e).astype(jnp.int32)
    raw_be = jnp.searchsorted(cb, jnp.arange(NB, dtype=jnp.int32),
                              side="right").astype(jnp.int32)
    block_expert = jnp.stack([jnp.minimum(raw_be, N_TILES - 1),
                              (raw_be < N_TILES).astype(jnp.int32)])

    hp = _sc_scatter_rows(dst, h, PADN)
    us3 = up_scale[:, None, :]
    dso = (down_scale * output_scale[:, None])[:, None, :]
    op = _grouped_ffn(block_expert, hp, up_w, down_w, us3, dso)
    tile_out = _sc_gather(tok2p, op, N_TOK)
    return x + tile_out


# speedup vs baseline: 1.8025x; 1.0396x over previous
import functools

import jax
import jax.numpy as jnp
from jax import lax
from jax.experimental import pallas as pl
from jax.experimental.pallas import tpu as pltpu
from jax.experimental.pallas import tpu_sc as plsc

D_MODEL = 1024
D_HID = 512
N_TILES = 64
N_TOK = 8192

TB = 256
PADN = N_TOK + N_TILES * TB
NB = PADN // TB

NC = 2
NS = 16
NW = NC * NS
_SC_MESH = dict(core_axis_name="c", subcore_axis_name="s",
                num_cores=NC, num_subcores=NS)

GCH = 32


def _sc_gather(gidx, src, n_rows, width=D_MODEL):
    n_per_w = n_rows // NW
    n_chunks = n_per_w // GCH

    @functools.partial(
        pl.kernel,
        mesh=plsc.VectorSubcoreMesh(**_SC_MESH),
        out_type=jax.ShapeDtypeStruct((n_rows, width), jnp.float32),
        scratch_types=[
            pltpu.VMEM((GCH,), jnp.int32),
            pltpu.VMEM((GCH, width), jnp.float32),
            pltpu.SemaphoreType.DMA,
        ],
    )
    def gather_k(gidx_hbm, src_hbm, out_hbm, idx_v, rows_v, sem):
        wid = lax.axis_index("s") * NC + lax.axis_index("c")
        base = wid * n_per_w

        def body(c, carry):
            off = base + c * GCH
            pltpu.sync_copy(gidx_hbm.at[pl.ds(off, GCH)], idx_v)
            pltpu.async_copy(src_hbm.at[idx_v], rows_v, sem).wait()
            pltpu.sync_copy(rows_v, out_hbm.at[pl.ds(off, GCH)])
            return carry

        lax.fori_loop(0, n_chunks, body, 0)

    return gather_k(gidx, src)


def _sc_scatter_rows(dst, src, n_out):
    n_per_w = N_TOK // NW
    n_chunks = n_per_w // GCH

    @functools.partial(
        pl.kernel,
        mesh=plsc.VectorSubcoreMesh(**_SC_MESH),
        out_type=jax.ShapeDtypeStruct((n_out, D_MODEL), jnp.float32),
        scratch_types=[
            pltpu.VMEM((GCH,), jnp.int32),
            pltpu.VMEM((GCH, D_MODEL), jnp.float32),
            pltpu.SemaphoreType.DMA,
        ],
    )
    def scatter_k(dst_hbm, src_hbm, out_hbm, idx_v, rows_v, sem):
        wid = lax.axis_index("s") * NC + lax.axis_index("c")
        base = wid * n_per_w

        def body(c, carry):
            off = base + c * GCH
            pltpu.sync_copy(dst_hbm.at[pl.ds(off, GCH)], idx_v)
            pltpu.sync_copy(src_hbm.at[pl.ds(off, GCH)], rows_v)
            pltpu.async_copy(rows_v, out_hbm.at[idx_v], sem).wait()
            return carry

        lax.fori_loop(0, n_chunks, body, 0)

    return scatter_k(dst, src)


def _ffn_block(be_ref, hp_ref, uw_ref, dw_ref, us_ref, dso_ref, out_ref,
               uws_ref, dws_ref):
    b = pl.program_id(0)
    valid = be_ref[1, b] == 1
    changed = jnp.logical_and(
        valid, jnp.logical_or(b == 0,
                              be_ref[0, b] != be_ref[0, jnp.maximum(b - 1, 0)]))

    @pl.when(changed)
    def _ternarize():
        uws_ref[...] = jnp.sign(uw_ref[0]).astype(jnp.bfloat16)
        dws_ref[...] = jnp.sign(dw_ref[0]).astype(jnp.bfloat16)

    @pl.when(valid)
    def _compute():
        hb = hp_ref[...].astype(jnp.bfloat16)
        hid = lax.dot_general(hb, uws_ref[...], (((1,), (1,)), ((), ())),
                              preferred_element_type=jnp.float32)
        hid = jnp.maximum(hid * us_ref[0], 0.0).astype(jnp.bfloat16)
        o = lax.dot_general(hid, dws_ref[...], (((1,), (1,)), ((), ())),
                            preferred_element_type=jnp.float32)
        out_ref[...] = o * dso_ref[0]


def _grouped_ffn(block_expert, hp, up_w, down_w, up_scale, dso):
    grid_spec = pltpu.PrefetchScalarGridSpec(
        num_scalar_prefetch=1,
        grid=(NB,),
        in_specs=[
            pl.BlockSpec((TB, D_MODEL),
                         lambda b, be: (jnp.where(be[1, b] == 1, b, 0), 0)),
            pl.BlockSpec((1, D_HID, D_MODEL), lambda b, be: (be[0, b], 0, 0)),
            pl.BlockSpec((1, D_MODEL, D_HID), lambda b, be: (be[0, b], 0, 0)),
            pl.BlockSpec((1, 1, D_HID), lambda b, be: (be[0, b], 0, 0)),
            pl.BlockSpec((1, 1, D_MODEL), lambda b, be: (be[0, b], 0, 0)),
        ],
        out_specs=pl.BlockSpec((TB, D_MODEL), lambda b, be: (b, 0)),
        scratch_shapes=[
            pltpu.VMEM((D_HID, D_MODEL), jnp.bfloat16),
            pltpu.VMEM((D_MODEL, D_HID), jnp.bfloat16),
        ],
    )
    return pl.pallas_call(
        _ffn_block,
        grid_spec=grid_spec,
        out_shape=jax.ShapeDtypeStruct((PADN, D_MODEL), jnp.float32),
    )(block_expert, hp, up_w, down_w, up_scale, dso)


def kernel(x, up_w, down_w, up_scale, down_scale, output_scale, norm_weight):
    h = x / jnp.sqrt(jnp.mean(x * x, axis=-1, keepdims=True) + 1e-6) * norm_weight
    sigs = jnp.sign(jnp.sign(up_w).sum(axis=1))
    scores = h @ sigs.T
    assign = jnp.argmax(scores, axis=-1).astype(jnp.int32)

    iota_n = jnp.arange(N_TOK, dtype=jnp.int32)
    oh = (assign[:, None] == jnp.arange(N_TILES, dtype=jnp.int32)[None, :])
    running = jnp.cumsum(oh.astype(jnp.int32), axis=0)
    counts = running[-1]
    ranks = jnp.sum(running * oh.astype(jnp.int32), axis=1) - 1
    blocks_per_e = (counts + TB - 1) // TB
    pad_sizes = blocks_per_e * TB
    pstart = jnp.concatenate([jnp.zeros((1,), jnp.int32),
                              jnp.cumsum(pad_sizes)[:-1].astype(jnp.int32)])
    dst = pstart[assign] + ranks
    tok2p = dst
    cb = jnp.cumsum(blocks_per_e).astype(jnp.int32)
    raw_be = jnp.searchsorted(cb, jnp.arange(NB, dtype=jnp.int32),
                              side="right").astype(jnp.int32)
    block_expert = jnp.stack([jnp.minimum(raw_be, N_TILES - 1),
                              (raw_be < N_TILES).astype(jnp.int32)])

    hp = _sc_scatter_rows(dst, h, PADN)
    us3 = up_scale[:, None, :]
    dso = (down_scale * output_scale[:, None])[:, None, :]
    op = _grouped_ffn(block_expert, hp, up_w, down_w, us3, dso)
    tile_out = _sc_gather(tok2p, op, N_TOK)
    return x + tile_out


# speedup vs baseline: 1.8045x; 1.0011x over previous
import functools

import jax
import jax.numpy as jnp
from jax import lax
from jax.experimental import pallas as pl
from jax.experimental.pallas import tpu as pltpu
from jax.experimental.pallas import tpu_sc as plsc

D_MODEL = 1024
D_HID = 512
N_TILES = 64
N_TOK = 8192

TB = 256
PADN = N_TOK + N_TILES * TB
NB = PADN // TB

NC = 2
NS = 16
NW = NC * NS
_SC_MESH = dict(core_axis_name="c", subcore_axis_name="s",
                num_cores=NC, num_subcores=NS)

GCH = 32


def _sc_gather(gidx, src, n_rows, width=D_MODEL):
    n_per_w = n_rows // NW
    n_chunks = n_per_w // GCH

    @functools.partial(
        pl.kernel,
        mesh=plsc.VectorSubcoreMesh(**_SC_MESH),
        out_type=jax.ShapeDtypeStruct((n_rows, width), jnp.float32),
        scratch_types=[
            pltpu.VMEM((GCH,), jnp.int32),
            pltpu.VMEM((GCH, width), jnp.float32),
            pltpu.SemaphoreType.DMA,
        ],
    )
    def gather_k(gidx_hbm, src_hbm, out_hbm, idx_v, rows_v, sem):
        wid = lax.axis_index("s") * NC + lax.axis_index("c")
        base = wid * n_per_w

        def body(c, carry):
            off = base + c * GCH
            pltpu.sync_copy(gidx_hbm.at[pl.ds(off, GCH)], idx_v)
            pltpu.async_copy(src_hbm.at[idx_v], rows_v, sem).wait()
            pltpu.sync_copy(rows_v, out_hbm.at[pl.ds(off, GCH)])
            return carry

        lax.fori_loop(0, n_chunks, body, 0)

    return gather_k(gidx, src)


def _sc_scatter_rows(dst, src, n_out):
    n_per_w = N_TOK // NW
    n_chunks = n_per_w // GCH

    @functools.partial(
        pl.kernel,
        mesh=plsc.VectorSubcoreMesh(**_SC_MESH),
        out_type=jax.ShapeDtypeStruct((n_out, D_MODEL), jnp.float32),
        scratch_types=[
            pltpu.VMEM((GCH,), jnp.int32),
            pltpu.VMEM((GCH, D_MODEL), jnp.float32),
            pltpu.SemaphoreType.DMA,
        ],
    )
    def scatter_k(dst_hbm, src_hbm, out_hbm, idx_v, rows_v, sem):
        wid = lax.axis_index("s") * NC + lax.axis_index("c")
        base = wid * n_per_w

        def body(c, carry):
            off = base + c * GCH
            pltpu.sync_copy(dst_hbm.at[pl.ds(off, GCH)], idx_v)
            pltpu.sync_copy(src_hbm.at[pl.ds(off, GCH)], rows_v)
            pltpu.async_copy(rows_v, out_hbm.at[idx_v], sem).wait()
            return carry

        lax.fori_loop(0, n_chunks, body, 0)

    return scatter_k(dst, src)


def _ffn_block(be_ref, hp_ref, uw_ref, dw_ref, us_ref, dso_ref, out_ref,
               uws_ref, dws_ref):
    b = pl.program_id(0)
    valid = be_ref[1, b] == 1
    changed = jnp.logical_and(
        valid, jnp.logical_or(b == 0,
                              be_ref[0, b] != be_ref[0, jnp.maximum(b - 1, 0)]))

    @pl.when(changed)
    def _ternarize():
        uws_ref[...] = jnp.sign(uw_ref[0]).astype(jnp.bfloat16)
        dws_ref[...] = jnp.sign(dw_ref[0]).astype(jnp.bfloat16)

    @pl.when(valid)
    def _compute():
        hb = hp_ref[...].astype(jnp.bfloat16)
        hid = lax.dot_general(hb, uws_ref[...], (((1,), (1,)), ((), ())),
                              preferred_element_type=jnp.float32)
        hid = jnp.maximum(hid * us_ref[0], 0.0).astype(jnp.bfloat16)
        o = lax.dot_general(hid, dws_ref[...], (((1,), (1,)), ((), ())),
                            preferred_element_type=jnp.float32)
        out_ref[...] = o * dso_ref[0]


def _grouped_ffn(block_expert, hp, up_w, down_w, up_scale, dso):
    grid_spec = pltpu.PrefetchScalarGridSpec(
        num_scalar_prefetch=1,
        grid=(NB,),
        in_specs=[
            pl.BlockSpec((TB, D_MODEL),
                         lambda b, be: (jnp.where(be[1, b] == 1, b, 0), 0)),
            pl.BlockSpec((1, D_HID, D_MODEL), lambda b, be: (be[0, b], 0, 0)),
            pl.BlockSpec((1, D_MODEL, D_HID), lambda b, be: (be[0, b], 0, 0)),
            pl.BlockSpec((1, 1, D_HID), lambda b, be: (be[0, b], 0, 0)),
            pl.BlockSpec((1, 1, D_MODEL), lambda b, be: (be[0, b], 0, 0)),
        ],
        out_specs=pl.BlockSpec((TB, D_MODEL), lambda b, be: (b, 0)),
        scratch_shapes=[
            pltpu.VMEM((D_HID, D_MODEL), jnp.bfloat16),
            pltpu.VMEM((D_MODEL, D_HID), jnp.bfloat16),
        ],
    )
    return pl.pallas_call(
        _ffn_block,
        grid_spec=grid_spec,
        out_shape=jax.ShapeDtypeStruct((PADN, D_MODEL), jnp.float32),
    )(block_expert, hp, up_w, down_w, up_scale, dso)


def kernel(x, up_w, down_w, up_scale, down_scale, output_scale, norm_weight):
    h = x / jnp.sqrt(jnp.mean(x * x, axis=-1, keepdims=True) + 1e-6) * norm_weight
    sigs = jnp.sign(jnp.sign(up_w).sum(axis=1))
    scores = h @ sigs.T
    assign = jnp.argmax(scores, axis=-1).astype(jnp.int32)

    oh = (assign[:, None] == jnp.arange(N_TILES, dtype=jnp.int32)[None, :])
    running = jnp.cumsum(oh.astype(jnp.int32), axis=0)
    counts = running[-1]
    ranks = jnp.sum(running * oh.astype(jnp.int32), axis=1) - 1
    blocks_per_e = (counts + TB - 1) // TB
    pad_sizes = blocks_per_e * TB
    pstart = jnp.concatenate([jnp.zeros((1,), jnp.int32),
                              jnp.cumsum(pad_sizes)[:-1].astype(jnp.int32)])
    dst = pstart[assign] + ranks
    tok2p = dst
    cb = jnp.cumsum(blocks_per_e).astype(jnp.int32)
    raw_be = jnp.searchsorted(cb, jnp.arange(NB, dtype=jnp.int32),
                              side="right").astype(jnp.int32)
    block_expert = jnp.stack([jnp.minimum(raw_be, N_TILES - 1),
                              (raw_be < N_TILES).astype(jnp.int32)])

    hp = _sc_scatter_rows(dst, h, PADN)
    us3 = up_scale[:, None, :]
    dso = (down_scale * output_scale[:, None])[:, None, :]
    op = _grouped_ffn(block_expert, hp, up_w, down_w, us3, dso)
    tile_out = _sc_gather(tok2p, op, N_TOK)
    return x + tile_out
